# Initial kernel scaffold; baseline (speedup 1.0000x reference)
import jax
import jax.numpy as jnp
from jax.experimental import pallas as pl


def kernel(input, target, noise_samples, logprob_noise, W, b):
    raise NotImplementedError("write your pallas kernel here")


# speedup vs baseline: 410.4483x; 410.4483x over previous
import functools
import math

import jax
import jax.numpy as jnp
from jax import lax
from jax.experimental import pallas as pl
from jax.experimental.pallas import tpu as pltpu
from jax.experimental.pallas import tpu_sc as plsc

V = 100000
D = 32
B = 1024
L = 20
K_NOISE = 100
N = B * L
NORM = float(math.log(V))

_NC = 2
_NS = 16
_NW = _NC * _NS
_PER_W = N // _NW
_CHUNK = 128
_NCHUNK = _PER_W // _CHUNK


def _sc_body(tgt_hbm, ns_hbm, w_hbm, lpn_hbm, b_hbm,
             g_out, tn_out, tb_out, wn_out, nn_out, nb_out,
             idx_v, rows_v, tn_v, tb_v, nidx_v, nrows_v, nn_v, nb_v,
             sem, nsem):
    wid = lax.axis_index("s") * _NC + lax.axis_index("c")
    base = wid * _PER_W
    pltpu.sync_copy(tgt_hbm.at[pl.ds(base, _PER_W)], idx_v)
    for j in range(_NCHUNK):
        sl = pl.ds(j * _CHUNK, _CHUNK)
        pltpu.async_copy(w_hbm.at[idx_v.at[sl]], rows_v.at[sl], sem).wait()
        pltpu.async_copy(lpn_hbm.at[idx_v.at[sl]], tn_v.at[sl], sem).wait()
        pltpu.async_copy(b_hbm.at[idx_v.at[sl]], tb_v.at[sl], sem).wait()
    pltpu.sync_copy(rows_v, g_out.at[pl.ds(base, _PER_W)])
    pltpu.sync_copy(tn_v, tn_out.at[pl.ds(base, _PER_W)])
    pltpu.sync_copy(tb_v, tb_out.at[pl.ds(base, _PER_W)])

    @pl.when(wid == 0)
    def _():
        pltpu.sync_copy(ns_hbm, nidx_v)
        pltpu.async_copy(w_hbm.at[nidx_v], nrows_v, nsem).wait()
        pltpu.async_copy(lpn_hbm.at[nidx_v], nn_v, nsem).wait()
        pltpu.async_copy(b_hbm.at[nidx_v], nb_v, nsem).wait()
        pltpu.sync_copy(nrows_v, wn_out)
        pltpu.sync_copy(nn_v, nn_out)
        pltpu.sync_copy(nb_v, nb_out)


@functools.cache
def _sc_gather_fn():
    return pl.kernel(
        _sc_body,
        mesh=plsc.VectorSubcoreMesh(
            core_axis_name="c", subcore_axis_name="s", num_cores=_NC),
        out_type=[
            jax.ShapeDtypeStruct((N, D), jnp.float32),
            jax.ShapeDtypeStruct((N,), jnp.float32),
            jax.ShapeDtypeStruct((N,), jnp.float32),
            jax.ShapeDtypeStruct((K_NOISE, D), jnp.float32),
            jax.ShapeDtypeStruct((K_NOISE,), jnp.float32),
            jax.ShapeDtypeStruct((K_NOISE,), jnp.float32),
        ],
        scratch_types=[
            pltpu.VMEM((_PER_W,), jnp.int32),
            pltpu.VMEM((_PER_W, D), jnp.float32),
            pltpu.VMEM((_PER_W,), jnp.float32),
            pltpu.VMEM((_PER_W,), jnp.float32),
            pltpu.VMEM((K_NOISE,), jnp.int32),
            pltpu.VMEM((K_NOISE, D), jnp.float32),
            pltpu.VMEM((K_NOISE,), jnp.float32),
            pltpu.VMEM((K_NOISE,), jnp.float32),
            pltpu.SemaphoreType.DMA,
            pltpu.SemaphoreType.DMA,
        ],
        compiler_params=pltpu.CompilerParams(use_tc_tiling_on_sc=False),
    )


_BB = 128
_GRID = B // _BB
_KP = 128


def _tc_body(x_ref, g_ref, tn_ref, tb_ref, wnt_ref, nn_ref, nb_ref, o_ref):
    x3 = x_ref[...]
    g3 = g_ref[...]
    tm = jnp.sum(x3 * g3, axis=2) + tb_ref[...] - NORM
    et = jnp.exp(tm)
    pt = et / (et + K_NOISE * jnp.exp(tn_ref[...]))
    pt = jnp.clip(pt, 1e-7, 1.0 - 1e-7)
    t_term = jnp.log(pt)

    x2 = x3.reshape(_BB * L, D)
    nm = jnp.dot(x2, wnt_ref[...], preferred_element_type=jnp.float32)
    nm = nm + nb_ref[...] - NORM
    en = jnp.exp(nm)
    pn = en / (en + K_NOISE * jnp.exp(nn_ref[...]))
    pn = jnp.clip(pn, 1e-7, 1.0 - 1e-7)
    ln = jnp.log(1.0 - pn)
    kmask = lax.broadcasted_iota(jnp.int32, (_BB * L, _KP), 1) < K_NOISE
    ln = jnp.where(kmask, ln, 0.0)
    ln3 = ln.reshape(_BB, L, _KP)
    total = t_term + jnp.sum(ln3, axis=2)
    o_ref[...] = jnp.sum(total, axis=1).reshape(1, 1, _BB)


_tc_call = pl.pallas_call(
    _tc_body,
    grid=(_GRID,),
    in_specs=[
        pl.BlockSpec((_BB, L, D), lambda g: (g, 0, 0)),
        pl.BlockSpec((_BB, L, D), lambda g: (g, 0, 0)),
        pl.BlockSpec((_BB, L), lambda g: (g, 0)),
        pl.BlockSpec((_BB, L), lambda g: (g, 0)),
        pl.BlockSpec((D, _KP), lambda g: (0, 0)),
        pl.BlockSpec((1, _KP), lambda g: (0, 0)),
        pl.BlockSpec((1, _KP), lambda g: (0, 0)),
    ],
    out_specs=pl.BlockSpec((1, 1, _BB), lambda g: (g, 0, 0)),
    out_shape=jax.ShapeDtypeStruct((_GRID, 1, _BB), jnp.float32),
)


def kernel(input, target, noise_samples, logprob_noise, W, b):
    tgt = target.reshape(N)
    g_rows, tn, tb, wn, nn, nb = _sc_gather_fn()(
        tgt, noise_samples, W, logprob_noise, b)
    g3 = g_rows.reshape(B, L, D)
    tn2 = tn.reshape(B, L)
    tb2 = tb.reshape(B, L)
    wnt = jnp.zeros((D, _KP), jnp.float32).at[:, :K_NOISE].set(wn.T)
    nn2 = jnp.zeros((1, _KP), jnp.float32).at[0, :K_NOISE].set(nn)
    nb2 = jnp.zeros((1, _KP), jnp.float32).at[0, :K_NOISE].set(nb)
    out2 = _tc_call(input, g3, tn2, tb2, wnt, nn2, nb2)
    return out2.reshape(B)


# speedup vs baseline: 447.4580x; 1.0902x over previous
import functools
import math

import jax
import jax.numpy as jnp
from jax import lax
from jax.experimental import pallas as pl
from jax.experimental.pallas import tpu as pltpu
from jax.experimental.pallas import tpu_sc as plsc

V = 100000
D = 32
B = 1024
L = 20
K_NOISE = 100
N = B * L
NORM = float(math.log(V))

_NC = 2
_NS = 16
_NW = _NC * _NS
_PER_W = N // _NW
_CHUNK = 128
_NCHUNK = _PER_W // _CHUNK


def _sc_body(tgt_hbm, ns_hbm, w_hbm, lpn_hbm, b_hbm,
             g_out, tn_out, tb_out, wn_out, nn_out, nb_out,
             idx_v, rows_v, tn_v, tb_v, nidx_v, nrows_v, nn_v, nb_v,
             sem, nsem):
    wid = lax.axis_index("s") * _NC + lax.axis_index("c")
    base = wid * _PER_W
    pltpu.sync_copy(tgt_hbm.at[pl.ds(base, _PER_W)], idx_v)
    copies = []
    for j in range(_NCHUNK):
        sl = pl.ds(j * _CHUNK, _CHUNK)
        copies.append(pltpu.async_copy(w_hbm.at[idx_v.at[sl]], rows_v.at[sl], sem))
        copies.append(pltpu.async_copy(lpn_hbm.at[idx_v.at[sl]], tn_v.at[sl], sem))
        copies.append(pltpu.async_copy(b_hbm.at[idx_v.at[sl]], tb_v.at[sl], sem))
    for c in copies:
        c.wait()
    pltpu.sync_copy(rows_v, g_out.at[pl.ds(base, _PER_W)])
    pltpu.sync_copy(tn_v, tn_out.at[pl.ds(base, _PER_W)])
    pltpu.sync_copy(tb_v, tb_out.at[pl.ds(base, _PER_W)])

    @pl.when(wid == 0)
    def _():
        pltpu.sync_copy(ns_hbm, nidx_v)
        c1 = pltpu.async_copy(w_hbm.at[nidx_v], nrows_v, nsem)
        c2 = pltpu.async_copy(lpn_hbm.at[nidx_v], nn_v, nsem)
        c3 = pltpu.async_copy(b_hbm.at[nidx_v], nb_v, nsem)
        c1.wait()
        c2.wait()
        c3.wait()
        pltpu.sync_copy(nrows_v, wn_out)
        pltpu.sync_copy(nn_v, nn_out)
        pltpu.sync_copy(nb_v, nb_out)


@functools.cache
def _sc_gather_fn():
    return pl.kernel(
        _sc_body,
        mesh=plsc.VectorSubcoreMesh(
            core_axis_name="c", subcore_axis_name="s", num_cores=_NC),
        out_type=[
            jax.ShapeDtypeStruct((N, D), jnp.float32),
            jax.ShapeDtypeStruct((N,), jnp.float32),
            jax.ShapeDtypeStruct((N,), jnp.float32),
            jax.ShapeDtypeStruct((K_NOISE, D), jnp.float32),
            jax.ShapeDtypeStruct((K_NOISE,), jnp.float32),
            jax.ShapeDtypeStruct((K_NOISE,), jnp.float32),
        ],
        scratch_types=[
            pltpu.VMEM((_PER_W,), jnp.int32),
            pltpu.VMEM((_PER_W, D), jnp.float32),
            pltpu.VMEM((_PER_W,), jnp.float32),
            pltpu.VMEM((_PER_W,), jnp.float32),
            pltpu.VMEM((K_NOISE,), jnp.int32),
            pltpu.VMEM((K_NOISE, D), jnp.float32),
            pltpu.VMEM((K_NOISE,), jnp.float32),
            pltpu.VMEM((K_NOISE,), jnp.float32),
            pltpu.SemaphoreType.DMA,
            pltpu.SemaphoreType.DMA,
        ],
        compiler_params=pltpu.CompilerParams(use_tc_tiling_on_sc=False),
    )


_BB = 128
_GRID = B // _BB
_KP = 128


LOG_LO = float(math.log(1e-7))
LOG_HI = float(math.log(1.0 - 1e-7))


def _tc_body(x_ref, g_ref, tn_ref, tbn_ref, wnt_ref, kc_ref, lkc_ref, nbn_ref, o_ref):
    x3 = x_ref[...]
    g3 = g_ref[...]
    tm = jnp.sum(x3 * g3, axis=2) + tbn_ref[...]
    et = jnp.exp(tm)
    kct = K_NOISE * jnp.exp(tn_ref[...])
    t_term = jnp.clip(tm - jnp.log(et + kct), LOG_LO, LOG_HI)

    x2 = x3.reshape(_BB * L, D)
    nm = jnp.dot(x2, wnt_ref[...], preferred_element_type=jnp.float32)
    nm = nm + nbn_ref[...]
    en = jnp.exp(nm)
    ln = jnp.clip(lkc_ref[...] - jnp.log(en + kc_ref[...]), LOG_LO, LOG_HI)
    kmask = lax.broadcasted_iota(jnp.int32, (_BB * L, _KP), 1) < K_NOISE
    ln = jnp.where(kmask, ln, 0.0)
    ln3 = ln.reshape(_BB, L, _KP)
    total = t_term + jnp.sum(ln3, axis=2)
    o_ref[...] = jnp.sum(total, axis=1).reshape(1, 1, _BB)


_tc_call = pl.pallas_call(
    _tc_body,
    grid=(_GRID,),
    in_specs=[
        pl.BlockSpec((_BB, L, D), lambda g: (g, 0, 0)),
        pl.BlockSpec((_BB, L, D), lambda g: (g, 0, 0)),
        pl.BlockSpec((_BB, L), lambda g: (g, 0)),
        pl.BlockSpec((_BB, L), lambda g: (g, 0)),
        pl.BlockSpec((D, _KP), lambda g: (0, 0)),
        pl.BlockSpec((1, _KP), lambda g: (0, 0)),
        pl.BlockSpec((1, _KP), lambda g: (0, 0)),
        pl.BlockSpec((1, _KP), lambda g: (0, 0)),
    ],
    out_specs=pl.BlockSpec((1, 1, _BB), lambda g: (g, 0, 0)),
    out_shape=jax.ShapeDtypeStruct((_GRID, 1, _BB), jnp.float32),
)


def kernel(input, target, noise_samples, logprob_noise, W, b):
    tgt = target.reshape(N)
    g_rows, tn, tb, wn, nn, nb = _sc_gather_fn()(
        tgt, noise_samples, W, logprob_noise, b)
    g3 = g_rows.reshape(B, L, D)
    tn2 = tn.reshape(B, L)
    tbn2 = tb.reshape(B, L) - NORM
    wnt = jnp.zeros((D, _KP), jnp.float32).at[:, :K_NOISE].set(wn.T)
    kc2 = K_NOISE * jnp.exp(jnp.zeros((1, _KP), jnp.float32).at[0, :K_NOISE].set(nn))
    lkc2 = jnp.log(kc2)
    nbn2 = jnp.zeros((1, _KP), jnp.float32).at[0, :K_NOISE].set(nb) - NORM
    out2 = _tc_call(input, g3, tn2, tbn2, wnt, kc2, lkc2, nbn2)
    return out2.reshape(B)
